# Initial kernel scaffold; baseline (speedup 1.0000x reference)
import jax
import jax.numpy as jnp
from jax.experimental import pallas as pl


def kernel(x, query, query_mask, batch, edge_index, query_pool, p_0, W_v, W_gate, W_u, W_q, W_x, W_h, w):
    raise NotImplementedError("write your pallas kernel here")


# speedup vs baseline: 20.0231x; 20.0231x over previous
import functools

import jax
import jax.numpy as jnp
from jax import lax
from jax.experimental import pallas as pl
from jax.experimental.pallas import tpu as pltpu
from jax.experimental.pallas import tpu_sc as plsc

N = 10000
E = 160000
D = 128
K = 4
Q = 8
S = 32
BN = 1000
GRID_N = N // BN
NP = 10112

NSC = 2
NTILE = 16
EP = E // NTILE
EPV = 10240
EPP = 10240
CH = 128
NCH = EPP // CH
RPT = NP // NTILE
ZR = 64
DC = D // 2
KH = 2 * K


def _init_ins_body(query_ref, qmask_ref, qpool_ref, wv_ref, wu_ref, out_ref):
    q = qpool_ref[...]
    query = query_ref[...]
    qm = qmask_ref[...]
    wu = wu_ref[...]
    i_prev = jnp.zeros((Q, D), jnp.float32)
    for k in range(K):
        cf = jnp.concatenate([i_prev, q, q * i_prev, q - i_prev], axis=1)
        qk = jnp.dot(cf, wv_ref[k], preferred_element_type=jnp.float32)
        t = qk[:, None, :] * query
        att = jnp.sum(t * wu, axis=2, keepdims=True)
        att = jnp.where(qm == 0, -1000000000.0, att)
        att = att - jnp.max(att, axis=1, keepdims=True)
        e = jnp.exp(att)
        u = e / jnp.sum(e, axis=1, keepdims=True)
        wq = u * query * qm
        ik = jnp.sum(wq, axis=1)
        out_ref[:, D * k:D * (k + 1)] = ik
        i_prev = ik


def _init_ins(query, qmask, qpool, W_v, wu3):
    return pl.pallas_call(
        _init_ins_body,
        out_shape=jax.ShapeDtypeStruct((Q, K * D), jnp.float32),
    )(query, qmask, qpool, W_v, wu3)


def _msg_body(h_ref, wx_ref, ins_ref, batch_ref, p_ref, out_ref):
    th = jnp.dot(h_ref[...], wx_ref[...], preferred_element_type=jnp.float32)
    b = batch_ref[...]
    oh = (b == lax.broadcasted_iota(jnp.int32, (BN, Q), 1)).astype(jnp.float32)
    ins = jnp.dot(oh, ins_ref[...], preferred_element_type=jnp.float32)
    p = p_ref[...]
    for k in range(K):
        out_ref[k] = jnp.maximum(ins[:, D * k:D * (k + 1)] * th, 0.0) * p


def _msgs(h, wx, ins, batch2, p):
    return pl.pallas_call(
        _msg_body,
        grid=(GRID_N,),
        in_specs=[
            pl.BlockSpec((BN, D), lambda i: (i, 0)),
            pl.BlockSpec((D, D), lambda i: (0, 0)),
            pl.BlockSpec((Q, K * D), lambda i: (0, 0)),
            pl.BlockSpec((BN, 1), lambda i: (i, 0)),
            pl.BlockSpec((BN, 1), lambda i: (i, 0)),
        ],
        out_specs=pl.BlockSpec((K, BN, D), lambda i: (0, i, 0)),
        out_shape=jax.ShapeDtypeStruct((K, N, D), jnp.float32),
    )(h, wx, ins, batch2, p)


SEG = 1280
NSEG = EPV // SEG
TAIL = EP - (NSEG - 1) * SEG


def _edge_body(wmsg_hbm, src_hbm, dst_hbm, out_hbm,
               src_sv, srck, dst2d, rows0, rows1, zbuf, agg_sh, sem0, sem1):
    rows = (rows0, rows1)
    sems = (sem0, sem1)
    c = lax.axis_index("c")
    s = lax.axis_index("s")
    ebase = s * EP
    zero16 = jnp.zeros((16,), jnp.int32)
    dumm16 = jnp.full((16,), N, jnp.int32)
    zrow = jnp.zeros((16,), jnp.float32)

    def zf(i, carry):
        for j in range(D // 16):
            zbuf[i, pl.ds(j * 16, 16)] = zrow
        return carry

    lax.fori_loop(0, ZR, zf, 0)

    rbase = s * RPT
    for kk in range(K // NSC):
        k = c * (K // NSC) + kk
        koff = k * N

        zcps = [pltpu.async_copy(zbuf, agg_sh.at[pl.ds(rbase + j * ZR, ZR)],
                                 sem0)
                for j in range(RPT // ZR)]
        rem = RPT - (RPT // ZR) * ZR
        if rem:
            zcps.append(pltpu.async_copy(
                zbuf.at[pl.ds(0, rem)],
                agg_sh.at[pl.ds(rbase + (RPT // ZR) * ZR, rem)], sem0))
        for zc in zcps:
            zc.wait()
        plsc.subcore_barrier()

        for g in range(NSEG):
            n_real = SEG if g < NSEG - 1 else TAIL
            pltpu.sync_copy(src_hbm.at[pl.ds(ebase + g * SEG, n_real)],
                            src_sv.at[pl.ds(0, n_real)])
            if n_real < SEG:
                def prefs(i, carry):
                    src_sv[pl.ds(n_real + i * 16, 16)] = zero16
                    return carry
                lax.fori_loop(0, (SEG - n_real) // 16, prefs, 0)

            def bk(i, carry):
                srck[pl.ds(i * 16, 16)] = src_sv[pl.ds(i * 16, 16)] + koff
                return carry

            lax.fori_loop(0, SEG // 16, bk, 0)

            pltpu.sync_copy(dst_hbm.at[pl.ds(ebase + g * SEG, n_real)],
                            src_sv.at[pl.ds(0, n_real)])
            if n_real < SEG:
                def prefd(i, carry):
                    src_sv[pl.ds(n_real + i * 16, 16)] = dumm16
                    return carry
                lax.fori_loop(0, (SEG - n_real) // 16, prefd, 0)

            def cpd(j, carry):
                for l in range(CH // 16):
                    dst2d[j, pl.ds(l * 16, 16)] = src_sv[pl.ds(j * CH + l * 16, 16)]
                return carry

            lax.fori_loop(0, SEG // CH, cpd, 0)

            ncc = SEG // CH
            cps = []
            for j in range(ncc):
                cp = pltpu.async_copy(
                    wmsg_hbm.at[srck.at[pl.ds(j * CH, CH)]],
                    rows[j % 2], sems[j % 2])
                if j > 0:
                    cps[j - 1].wait()
                    pltpu.sync_copy(rows[(j - 1) % 2],
                                    agg_sh.at[dst2d.at[j - 1]], add=True)
                cps.append(cp)
            cps[ncc - 1].wait()
            pltpu.sync_copy(rows[(ncc - 1) % 2],
                            agg_sh.at[dst2d.at[ncc - 1]], add=True)

        plsc.subcore_barrier()
        pltpu.sync_copy(agg_sh.at[pl.ds(rbase, RPT)],
                        out_hbm.at[pl.ds(k * NP + rbase, RPT)])
        plsc.subcore_barrier()


_edge_sc = functools.partial(
    pl.kernel,
    mesh=plsc.VectorSubcoreMesh(core_axis_name="c", subcore_axis_name="s"),
    out_type=jax.ShapeDtypeStruct((K * NP, D), jnp.float32),
    scratch_types=[
        pltpu.VMEM((SEG,), jnp.int32),
        pltpu.VMEM((SEG,), jnp.int32),
        pltpu.VMEM((SEG // CH, CH), jnp.int32),
        pltpu.VMEM((CH, D), jnp.float32),
        pltpu.VMEM((CH, D), jnp.float32),
        pltpu.VMEM((ZR, D), jnp.float32),
        pltpu.VMEM_SHARED((NP, D), jnp.float32),
        pltpu.SemaphoreType.DMA,
        pltpu.SemaphoreType.DMA,
    ],
)(_edge_body)


def _edge_pass(wmsg4, src1d, dst1d):
    wmsg_flat = wmsg4.reshape(K * N, D)
    aggf = _edge_sc(wmsg_flat, src1d, dst1d)
    return aggf.reshape(K, NP, D)


def _comb_body(h_ref, agg_ref, wh_ref, w_ref, hnew_ref, sc_ref):
    h = h_ref[...]
    parts = [h] + [agg_ref[k] for k in range(K)]
    comb = jnp.concatenate(parts, axis=1)
    hn = jnp.maximum(jnp.dot(comb, wh_ref[...], preferred_element_type=jnp.float32), 0.0)
    hnew_ref[...] = hn
    sc_ref[...] = jnp.sum(hn * w_ref[...], axis=1, keepdims=True)


def _combine(h, agg4, wh, w2):
    return pl.pallas_call(
        _comb_body,
        grid=(GRID_N,),
        in_specs=[
            pl.BlockSpec((BN, D), lambda i: (i, 0)),
            pl.BlockSpec((K, BN, D), lambda i: (0, i, 0)),
            pl.BlockSpec(((K + 1) * D, D), lambda i: (0, 0)),
            pl.BlockSpec((1, D), lambda i: (0, 0)),
        ],
        out_specs=[
            pl.BlockSpec((BN, D), lambda i: (i, 0)),
            pl.BlockSpec((BN, 1), lambda i: (i, 0)),
        ],
        out_shape=[
            jax.ShapeDtypeStruct((N, D), jnp.float32),
            jax.ShapeDtypeStruct((N, 1), jnp.float32),
        ],
    )(h, agg4, wh, w2)


def _softmax_body(s_ref, out_ref):
    sc = s_ref[...]
    m = jnp.max(sc)
    e = jnp.exp(sc - m)
    out_ref[...] = e / jnp.sum(e)


def _softmax(scores):
    return pl.pallas_call(
        _softmax_body,
        out_shape=jax.ShapeDtypeStruct((N, 1), jnp.float32),
    )(scores)


def _he_body(h_ref, batch_ref, p0_ref, out_ref):
    i = pl.program_id(0)

    @pl.when(i == 0)
    def _():
        out_ref[...] = jnp.zeros((Q, D), jnp.float32)

    seed = (p0_ref[...] > 0).astype(jnp.float32)
    hs = h_ref[...] * seed
    b = batch_ref[...]
    oh = (b == lax.broadcasted_iota(jnp.int32, (BN, Q), 1)).astype(jnp.float32)
    out_ref[...] += lax.dot_general(
        oh, hs, (((0,), (0,)), ((), ())), preferred_element_type=jnp.float32)


def _he(h, batch2, p0c):
    return pl.pallas_call(
        _he_body,
        grid=(GRID_N,),
        in_specs=[
            pl.BlockSpec((BN, D), lambda i: (i, 0)),
            pl.BlockSpec((BN, 1), lambda i: (i, 0)),
            pl.BlockSpec((BN, 1), lambda i: (i, 0)),
        ],
        out_specs=pl.BlockSpec((Q, D), lambda i: (0, 0)),
        out_shape=jax.ShapeDtypeStruct((Q, D), jnp.float32),
    )(h, batch2, p0c)


def _gate_body(ins_ref, he_ref, wq_ref, wg_ref, out_ref):
    he = he_ref[...]
    for k in range(K):
        ik = ins_ref[:, D * k:D * (k + 1)]
        cf = jnp.concatenate([ik, he, ik - he, ik * he], axis=1)
        tr = jnp.dot(cf, wq_ref[...], preferred_element_type=jnp.float32)
        g = jax.nn.sigmoid(jnp.dot(cf, wg_ref[k], preferred_element_type=jnp.float32))
        out_ref[:, D * k:D * (k + 1)] = (1 - g) * ik + g * tr


def _gate(ins, he, W_q, W_gate):
    return pl.pallas_call(
        _gate_body,
        out_shape=jax.ShapeDtypeStruct((Q, K * D), jnp.float32),
    )(ins, he, W_q, W_gate)


def _layer(h, p, p1d, ins, batch2, src1d, dst1d, wx, wh, w2, need_p):
    wmsg4 = _msgs(h, wx, ins, batch2, p)
    agg4 = _edge_pass(wmsg4, src1d, dst1d)
    h_new, scores = _combine(h, agg4, wh, w2)
    if need_p:
        p_new = _softmax(scores)
        return h_new, p_new, p_new.reshape(N)
    return h_new, None, None


def kernel(x, query, query_mask, batch, edge_index, query_pool, p_0,
           W_v, W_gate, W_u, W_q, W_x, W_h, w):
    batch2 = batch.astype(jnp.int32).reshape(N, 1)
    ei = edge_index.astype(jnp.int32)
    src1d, dst1d = ei[0], ei[1]
    p0c = p_0.reshape(N, 1)
    wu3 = W_u.reshape(1, 1, D)
    w2 = w.reshape(1, D)

    ins = _init_ins(query, query_mask, query_pool, W_v, wu3)

    h1, p1, p1f = _layer(x, p0c, p_0, ins, batch2, src1d, dst1d, W_x[0], W_h[0], w2, True)
    h2, _, _ = _layer(h1, p1, p1f, ins, batch2, src1d, dst1d, W_x[1], W_h[1], w2, False)
    he = _he(h2, batch2, p0c)
    ins = _gate(ins, he, W_q, W_gate)

    h3, p3, p3f = _layer(h2, p0c, p_0, ins, batch2, src1d, dst1d, W_x[0], W_h[0], w2, True)
    _, p4, _ = _layer(h3, p3, p3f, ins, batch2, src1d, dst1d, W_x[1], W_h[1], w2, True)
    return p4.reshape(N)


# speedup vs baseline: 20.0511x; 1.0014x over previous
import functools

import jax
import jax.numpy as jnp
from jax import lax
from jax.experimental import pallas as pl
from jax.experimental.pallas import tpu as pltpu
from jax.experimental.pallas import tpu_sc as plsc

N = 10000
E = 160000
D = 128
K = 4
Q = 8
S = 32
BN = 1000
GRID_N = N // BN
NP = 10112

NSC = 2
NTILE = 16
EP = E // NTILE
EPV = 10240
EPP = 10240
CH = 128
NCH = EPP // CH
RPT = NP // NTILE
ZR = 64
DC = D // 2
KH = 2 * K


def _init_ins_body(query_ref, qmask_ref, qpool_ref, wv_ref, wu_ref, out_ref):
    q = qpool_ref[...]
    query = query_ref[...]
    qm = qmask_ref[...]
    wu = wu_ref[...]
    i_prev = jnp.zeros((Q, D), jnp.float32)
    for k in range(K):
        cf = jnp.concatenate([i_prev, q, q * i_prev, q - i_prev], axis=1)
        qk = jnp.dot(cf, wv_ref[k], preferred_element_type=jnp.float32)
        t = qk[:, None, :] * query
        att = jnp.sum(t * wu, axis=2, keepdims=True)
        att = jnp.where(qm == 0, -1000000000.0, att)
        att = att - jnp.max(att, axis=1, keepdims=True)
        e = jnp.exp(att)
        u = e / jnp.sum(e, axis=1, keepdims=True)
        wq = u * query * qm
        ik = jnp.sum(wq, axis=1)
        out_ref[:, D * k:D * (k + 1)] = ik
        i_prev = ik


def _init_ins(query, qmask, qpool, W_v, wu3):
    return pl.pallas_call(
        _init_ins_body,
        out_shape=jax.ShapeDtypeStruct((Q, K * D), jnp.float32),
    )(query, qmask, qpool, W_v, wu3)


def _msg_body(h_ref, wx_ref, ins_ref, batch_ref, p_ref, out_ref):
    th = jnp.dot(h_ref[...], wx_ref[...], preferred_element_type=jnp.float32)
    b = batch_ref[...]
    oh = (b == lax.broadcasted_iota(jnp.int32, (BN, Q), 1)).astype(jnp.float32)
    ins = jnp.dot(oh, ins_ref[...], preferred_element_type=jnp.float32)
    p = p_ref[...]
    for k in range(K):
        out_ref[k] = jnp.maximum(ins[:, D * k:D * (k + 1)] * th, 0.0) * p


def _msgs(h, wx, ins, batch2, p):
    return pl.pallas_call(
        _msg_body,
        grid=(GRID_N,),
        in_specs=[
            pl.BlockSpec((BN, D), lambda i: (i, 0)),
            pl.BlockSpec((D, D), lambda i: (0, 0)),
            pl.BlockSpec((Q, K * D), lambda i: (0, 0)),
            pl.BlockSpec((BN, 1), lambda i: (i, 0)),
            pl.BlockSpec((BN, 1), lambda i: (i, 0)),
        ],
        out_specs=pl.BlockSpec((K, BN, D), lambda i: (0, i, 0)),
        out_shape=jax.ShapeDtypeStruct((K, N, D), jnp.float32),
    )(h, wx, ins, batch2, p)


SEG = 1280
NSEG = EPV // SEG
TAIL = EP - (NSEG - 1) * SEG


def _edge_body(wmsg_hbm, src_hbm, dst_hbm, out_hbm,
               src_sv, srck, dst2d, rows0, rows1, zbuf, agg_sh, sem0, sem1):
    rows = (rows0, rows1)
    sems = (sem0, sem1)
    c = lax.axis_index("c")
    s = lax.axis_index("s")
    ebase = s * EP
    zero16 = jnp.zeros((16,), jnp.int32)
    dumm16 = jnp.full((16,), N, jnp.int32)
    zrow = jnp.zeros((16,), jnp.float32)

    def zf(i, carry):
        for j in range(D // 16):
            zbuf[i, pl.ds(j * 16, 16)] = zrow
        return carry

    lax.fori_loop(0, ZR, zf, 0)

    rbase = s * RPT
    for kk in range(K // NSC):
        k = c * (K // NSC) + kk
        koff = k * N

        zcps = [pltpu.async_copy(zbuf, agg_sh.at[pl.ds(rbase + j * ZR, ZR)],
                                 sem0)
                for j in range(RPT // ZR)]
        rem = RPT - (RPT // ZR) * ZR
        if rem:
            zcps.append(pltpu.async_copy(
                zbuf.at[pl.ds(0, rem)],
                agg_sh.at[pl.ds(rbase + (RPT // ZR) * ZR, rem)], sem0))
        for zc in zcps:
            zc.wait()
        plsc.subcore_barrier()

        for g in range(NSEG):
            n_real = SEG if g < NSEG - 1 else TAIL
            pltpu.sync_copy(src_hbm.at[pl.ds(ebase + g * SEG, n_real)],
                            src_sv.at[pl.ds(0, n_real)])
            if n_real < SEG:
                def prefs(i, carry):
                    src_sv[pl.ds(n_real + i * 16, 16)] = zero16
                    return carry
                lax.fori_loop(0, (SEG - n_real) // 16, prefs, 0)

            def bk(i, carry):
                srck[pl.ds(i * 16, 16)] = src_sv[pl.ds(i * 16, 16)] + koff
                return carry

            lax.fori_loop(0, SEG // 16, bk, 0)

            pltpu.sync_copy(dst_hbm.at[pl.ds(ebase + g * SEG, n_real)],
                            src_sv.at[pl.ds(0, n_real)])
            if n_real < SEG:
                def prefd(i, carry):
                    src_sv[pl.ds(n_real + i * 16, 16)] = dumm16
                    return carry
                lax.fori_loop(0, (SEG - n_real) // 16, prefd, 0)

            def cpd(j, carry):
                for l in range(CH // 16):
                    dst2d[j, pl.ds(l * 16, 16)] = src_sv[pl.ds(j * CH + l * 16, 16)]
                return carry

            lax.fori_loop(0, SEG // CH, cpd, 0)

            ncc = SEG // CH
            cps = []
            for j in range(ncc):
                cp = pltpu.async_copy(
                    wmsg_hbm.at[srck.at[pl.ds(j * CH, CH)]],
                    rows[j % 2], sems[j % 2])
                if j > 0:
                    cps[j - 1].wait()
                    pltpu.sync_copy(rows[(j - 1) % 2],
                                    agg_sh.at[dst2d.at[j - 1]], add=True)
                cps.append(cp)
            cps[ncc - 1].wait()
            pltpu.sync_copy(rows[(ncc - 1) % 2],
                            agg_sh.at[dst2d.at[ncc - 1]], add=True)

        plsc.subcore_barrier()
        pltpu.sync_copy(agg_sh.at[pl.ds(rbase, RPT)],
                        out_hbm.at[pl.ds(k * NP + rbase, RPT)])
        plsc.subcore_barrier()


_edge_sc = functools.partial(
    pl.kernel,
    mesh=plsc.VectorSubcoreMesh(core_axis_name="c", subcore_axis_name="s"),
    out_type=jax.ShapeDtypeStruct((K * NP, D), jnp.float32),
    scratch_types=[
        pltpu.VMEM((SEG,), jnp.int32),
        pltpu.VMEM((SEG,), jnp.int32),
        pltpu.VMEM((SEG // CH, CH), jnp.int32),
        pltpu.VMEM((CH, D), jnp.float32),
        pltpu.VMEM((CH, D), jnp.float32),
        pltpu.VMEM((ZR, D), jnp.float32),
        pltpu.VMEM_SHARED((NP, D), jnp.float32),
        pltpu.SemaphoreType.DMA,
        pltpu.SemaphoreType.DMA,
    ],
)(_edge_body)


def _edge_pass(wmsg4, src1d, dst1d):
    wmsg_flat = wmsg4.reshape(K * N, D)
    aggf = _edge_sc(wmsg_flat, src1d, dst1d)
    return aggf.reshape(K, NP, D)


def _comb_body(h_ref, agg_ref, wh_ref, w_ref, hnew_ref, sc_ref):
    h = h_ref[...]
    parts = [h] + [agg_ref[k] for k in range(K)]
    comb = jnp.concatenate(parts, axis=1)
    hn = jnp.maximum(jnp.dot(comb, wh_ref[...], preferred_element_type=jnp.float32), 0.0)
    hnew_ref[...] = hn
    sc_ref[...] = jnp.sum(hn * w_ref[...], axis=1, keepdims=True)


def _combine(h, agg4, wh, w2):
    return pl.pallas_call(
        _comb_body,
        grid=(GRID_N,),
        in_specs=[
            pl.BlockSpec((BN, D), lambda i: (i, 0)),
            pl.BlockSpec((K, BN, D), lambda i: (0, i, 0)),
            pl.BlockSpec(((K + 1) * D, D), lambda i: (0, 0)),
            pl.BlockSpec((1, D), lambda i: (0, 0)),
        ],
        out_specs=[
            pl.BlockSpec((BN, D), lambda i: (i, 0)),
            pl.BlockSpec((BN, 1), lambda i: (i, 0)),
        ],
        out_shape=[
            jax.ShapeDtypeStruct((N, D), jnp.float32),
            jax.ShapeDtypeStruct((N, 1), jnp.float32),
        ],
    )(h, agg4, wh, w2)


def _softmax_body(s_ref, out_ref):
    sc = s_ref[...]
    m = jnp.max(sc)
    e = jnp.exp(sc - m)
    out_ref[...] = e / jnp.sum(e)


def _softmax(scores):
    return pl.pallas_call(
        _softmax_body,
        out_shape=jax.ShapeDtypeStruct((N, 1), jnp.float32),
    )(scores)


def _he_body(h_ref, batch_ref, p0_ref, out_ref):
    i = pl.program_id(0)

    @pl.when(i == 0)
    def _():
        out_ref[...] = jnp.zeros((Q, D), jnp.float32)

    seed = (p0_ref[...] > 0).astype(jnp.float32)
    hs = h_ref[...] * seed
    b = batch_ref[...]
    oh = (b == lax.broadcasted_iota(jnp.int32, (BN, Q), 1)).astype(jnp.float32)
    out_ref[...] += lax.dot_general(
        oh, hs, (((0,), (0,)), ((), ())), preferred_element_type=jnp.float32)


def _he(h, batch2, p0c):
    return pl.pallas_call(
        _he_body,
        grid=(GRID_N,),
        in_specs=[
            pl.BlockSpec((BN, D), lambda i: (i, 0)),
            pl.BlockSpec((BN, 1), lambda i: (i, 0)),
            pl.BlockSpec((BN, 1), lambda i: (i, 0)),
        ],
        out_specs=pl.BlockSpec((Q, D), lambda i: (0, 0)),
        out_shape=jax.ShapeDtypeStruct((Q, D), jnp.float32),
    )(h, batch2, p0c)


def _gate_body(ins_ref, he_ref, wq_ref, wg_ref, out_ref):
    he = he_ref[...]
    for k in range(K):
        ik = ins_ref[:, D * k:D * (k + 1)]
        cf = jnp.concatenate([ik, he, ik - he, ik * he], axis=1)
        tr = jnp.dot(cf, wq_ref[...], preferred_element_type=jnp.float32)
        g = jax.nn.sigmoid(jnp.dot(cf, wg_ref[k], preferred_element_type=jnp.float32))
        out_ref[:, D * k:D * (k + 1)] = (1 - g) * ik + g * tr


def _gate(ins, he, W_q, W_gate):
    return pl.pallas_call(
        _gate_body,
        out_shape=jax.ShapeDtypeStruct((Q, K * D), jnp.float32),
    )(ins, he, W_q, W_gate)


def _layer(h, p, ins, batch2, src1d, dst1d, wx, wh, w2, need_p):
    wmsg4 = _msgs(h, wx, ins, batch2, p)
    agg4 = _edge_pass(wmsg4, src1d, dst1d)
    h_new, scores = _combine(h, agg4, wh, w2)
    p_new = _softmax(scores) if need_p else None
    return h_new, p_new


def kernel(x, query, query_mask, batch, edge_index, query_pool, p_0,
           W_v, W_gate, W_u, W_q, W_x, W_h, w):
    batch2 = batch.astype(jnp.int32).reshape(N, 1)
    ei = edge_index.astype(jnp.int32)
    src1d, dst1d = ei[0], ei[1]
    p0c = p_0.reshape(N, 1)
    wu3 = W_u.reshape(1, 1, D)
    w2 = w.reshape(1, D)

    ins = _init_ins(query, query_mask, query_pool, W_v, wu3)

    h1, p1 = _layer(x, p0c, ins, batch2, src1d, dst1d, W_x[0], W_h[0], w2, True)
    h2, _ = _layer(h1, p1, ins, batch2, src1d, dst1d, W_x[1], W_h[1], w2, False)
    he = _he(h2, batch2, p0c)
    ins = _gate(ins, he, W_q, W_gate)

    h3, p3 = _layer(h2, p0c, ins, batch2, src1d, dst1d, W_x[0], W_h[0], w2, True)
    _, p4 = _layer(h3, p3, ins, batch2, src1d, dst1d, W_x[1], W_h[1], w2, True)
    return p4.reshape(N)
